# Initial kernel scaffold; baseline (speedup 1.0000x reference)
import jax
import jax.numpy as jnp
from jax.experimental import pallas as pl


def kernel(idxs, embedding, token_weights):
    raise NotImplementedError("write your pallas kernel here")


# speedup vs baseline: 9.4001x; 9.4001x over previous
import functools

import jax
import jax.numpy as jnp
from jax import lax
from jax.experimental import pallas as pl
from jax.experimental.pallas import tpu as pltpu
from jax.experimental.pallas import tpu_sc as plsc

NC = 2
NS = 16
NW = NC * NS
L = 16

BATCH = 4096
HIST = 200
HP = 208
CHUNK = HP // 2
EMBED = 32
B_PER_W = BATCH // NW


def _nbow_kernel(idxs_hbm, emb_hbm, tw_hbm, out_hbm,
                 idx_v, w_v, rows_v, out_slab, sem_e, sem_w):
    wid = lax.axis_index("s") * NC + lax.axis_index("c")
    base = wid * B_PER_W

    def row_body(i, _):
        pltpu.sync_copy(idxs_hbm.at[base + i], idx_v)

        cps = []
        for c in range(2):
            sl = pl.ds(c * CHUNK, CHUNK)
            cps.append(pltpu.async_copy(
                emb_hbm.at[idx_v.at[sl]], rows_v.at[sl], sem_e))
            cps.append(pltpu.async_copy(
                tw_hbm.at[idx_v.at[sl]], w_v.at[sl], sem_w))
        for cp in cps:
            cp.wait()

        for k in range(HP // L):
            sl = pl.ds(k * L, L)
            iv = idx_v[sl]
            w_v[sl] = jnp.where(iv != 0, w_v[sl], 0.0)

        def fma_body(b, carry):
            a0, a1 = carry
            wv = w_v[pl.ds(b * L, L)]
            for jj in range(L):
                j = b * L + jj
                ws = wv[jj]
                a0 = a0 + ws * rows_v[j, pl.ds(0, L)]
                a1 = a1 + ws * rows_v[j, pl.ds(L, L)]
            return (a0, a1)

        zero = jnp.zeros((L,), jnp.float32)
        a0, a1 = lax.fori_loop(0, HP // L, fma_body, (zero, zero))
        out_slab[i, pl.ds(0, L)] = a0
        out_slab[i, pl.ds(L, L)] = a1
        return 0

    lax.fori_loop(0, B_PER_W, row_body, 0)
    pltpu.sync_copy(out_slab, out_hbm.at[pl.ds(base, B_PER_W)])


@jax.jit
def kernel(idxs, embedding, token_weights):
    idxs_p = jnp.pad(idxs, ((0, 0), (0, HP - HIST)))

    mesh = plsc.VectorSubcoreMesh(core_axis_name="c", subcore_axis_name="s")
    k = functools.partial(
        pl.kernel,
        out_type=jax.ShapeDtypeStruct((BATCH, EMBED), jnp.float32),
        mesh=mesh,
        scratch_types=[
            pltpu.VMEM((HP,), jnp.int32),
            pltpu.VMEM((HP,), jnp.float32),
            pltpu.VMEM((HP, EMBED), jnp.float32),
            pltpu.VMEM((B_PER_W, EMBED), jnp.float32),
            pltpu.SemaphoreType.DMA,
            pltpu.SemaphoreType.DMA,
        ],
        compiler_params=pltpu.CompilerParams(use_tc_tiling_on_sc=False),
    )(_nbow_kernel)
    return k(idxs_p, embedding, token_weights)
